# baseline (speedup 1.0000x reference)
import jax
import jax.numpy as jnp
from jax import lax
from jax.experimental import pallas as pl
from jax.experimental.pallas import tpu as pltpu
from jax.experimental.pallas import tpu_sc as plsc

B = 16
M = 16384
C = 512
N = 2048
HALF = 1024
L = 16
CH = 64
NCH = HALF // CH
TR = 4 * CH
XFETCH = 1040
XSTRIDE = 1056

_SIZES_REM = [(1 << k) for k in range(5, -1, -1)]


def _sc_body(bidx0, ftab0, xyzt0, bidx1, ftab1, xyzt1, zf,
             f0, p0, m0, f1, p1, m1,
             bbuf, ring, ixbuf, xyb, xytmp, mbuf, zbuf,
             sem_in, semo0, semo1, sem_x, sem_misc):
    cax = lax.axis_index("c")
    sax = lax.axis_index("s")
    wid = cax * 16 + sax
    b = wid // 2
    h = wid % 2
    d0 = wid * HALF
    semos = (semo0, semo1)

    cz = pltpu.async_copy(zf, zbuf, sem_misc)
    cb = pltpu.async_copy(bidx0, bbuf, sem_misc)
    cz.wait()
    cb.wait()

    iota = lax.broadcasted_iota(jnp.int32, (L,), 0)

    def counts(bv):
        def step(i, carry):
            lo, hi = carry
            for u in range(4):
                x = bv[pl.ds((i * 4 + u) * L, L)]
                lo = lo + jnp.where(x < b, 1, 0).astype(jnp.int32)
                hi = hi + jnp.where(x <= b, 1, 0).astype(jnp.int32)
            return lo, hi
        z = jnp.zeros((L,), jnp.int32)
        lo, hi = lax.fori_loop(0, M // L // 4, step, (z, z))
        return jnp.sum(lo), jnp.sum(hi)

    seg0 = counts(bbuf)
    pltpu.sync_copy(bidx1, bbuf)
    seg1 = counts(bbuf)

    def fill_mask(v, mb):
        def step(i, carry):
            for u in range(4):
                j = i * 4 + u
                idx = j * L + iota
                mbuf[pl.ds(mb + j * L, L)] = jnp.where(
                    idx < v, 0.0, 1.0).astype(jnp.float32)
            return carry
        lax.fori_loop(0, HALF // L // 4, step, 0)

    def tiled_idx(i, g16):
        return ((i >> 3) << 5) + (g16 << 3) + (i & 7)

    deferred = []

    for si, (seg, ftab, xyzt, fout, pout, mout) in enumerate((
            (seg0, ftab0, xyzt0, f0, p0, m0),
            (seg1, ftab1, xyzt1, f1, p1, m1))):
        s0, e0 = seg
        v = jnp.clip(e0 - s0 - h * HALF, 0, HALF)
        srow = s0 + h * HALF
        rem = v & (CH - 1)
        nv = HALF - v

        sal = jnp.minimum((srow >> 3) << 3, M - XFETCH)
        sh = srow - sal
        xyz_ins = []
        for comp in range(3):
            a0 = pl.multiple_of(comp * M + sal, 8)
            d = pltpu.make_async_copy(
                xyzt.at[pl.ds(a0, XFETCH)],
                xytmp.at[pl.ds(comp * XSTRIDE, XFETCH)], sem_x)
            d.start()
            xyz_ins.append(d)

        seeds = []
        for t in range(16):
            j = t * L + iota
            seeds.append(tiled_idx(srow + (j >> 2), j & 3))

        def fire_chunk(c, slot):
            for t in range(16):
                ixbuf[pl.ds(slot * TR + t * L, L)] = seeds[t] + c * TR
            gs = []
            for hc in range(2):
                gs.append(pltpu.async_copy(
                    ftab.at[ixbuf.at[pl.ds(slot * TR + hc * 128, 128)]],
                    ring.at[pl.ds(slot * TR + hc * 128, 128)], sem_in))
            return gs

        for c in range(NCH):
            @pl.when(v >= (c + 1) * CH)
            def _(c=c):
                slot = c & 1
                if c >= 2:
                    pltpu.make_async_copy(
                        ring.at[pl.ds(slot * TR, TR)],
                        fout.at[pl.ds((d0 + (c - 2) * CH) * 4, TR)],
                        semos[slot]).wait()
                for g in fire_chunk(c, slot):
                    g.wait()
                pltpu.async_copy(
                    ring.at[pl.ds(slot * TR, TR)],
                    fout.at[pl.ds((d0 + c * CH) * 4, TR)], semos[slot])
        for slot in range(2):
            @pl.when(v >= (slot + 1) * CH)
            def _(slot=slot):
                pltpu.make_async_copy(
                    ring.at[pl.ds(slot * TR, TR)],
                    fout.at[pl.ds(d0 * 4, TR)], semos[slot]).wait()

        @pl.when(rem != 0)
        def _(v=v, rem=rem, srow=srow, fout=fout, ftab=ftab):
            cb0 = v - rem
            imax = srow + v - 1
            for t in range(16):
                j = t * L + iota
                i = jnp.minimum(srow + cb0 + (j >> 2), imax)
                ixbuf[pl.ds(t * L, L)] = tiled_idx(i, j & 3)
            for hc in range(2):
                pltpu.async_copy(
                    ftab.at[ixbuf.at[pl.ds(hc * 128, 128)]],
                    ring.at[pl.ds(hc * 128, 128)], sem_in).wait()
            tz = CH - rem
            for action in ("s", "w"):
                for size in _SIZES_REM:
                    k = size.bit_length()
                    zoff = (tz >> k) << k

                    @pl.when((tz & size) != 0)
                    def _(zoff=zoff, size=size):
                        d = pltpu.make_async_copy(
                            zf.at[pl.ds(0, 4 * size)],
                            ring.at[pl.ds(4 * (rem + zoff), 4 * size)],
                            sem_in)
                        d.start() if action == "s" else d.wait()
            pltpu.sync_copy(ring.at[pl.ds(0, TR)],
                            fout.at[pl.ds((d0 + cb0) * 4, TR)])

        def zfill(action, v=v, fout=fout):
            zstart = ((v + CH - 1) >> 6) << 6
            q = (HALF - zstart) >> 5

            def zstep(j, carry):
                d = pltpu.make_async_copy(
                    zbuf, fout.at[pl.ds((d0 + zstart) * 4 + j * 128, 128)],
                    sem_misc)
                d.start() if action == "s" else d.wait()
                return carry
            lax.fori_loop(0, q, zstep, 0)
        zfill("s")
        deferred.append(lambda zfill=zfill: zfill("w"))

        for d in xyz_ins:
            d.wait()
        for comp in range(3):
            tb = comp * XSTRIDE
            ob = (si * 3 + comp) * XFETCH

            def xstep(i, carry, tb=tb, ob=ob):
                xyb[pl.ds(ob + i * L, L)] = xytmp[pl.ds(tb + sh + i * L, L)]
                return carry
            lax.fori_loop(0, v >> 4, xstep, 0)
            fl = (v >> 4) << 4
            xv = xytmp[pl.ds(tb + sh + fl, L)]
            xyb[pl.ds(ob + fl, L)] = jnp.where(
                iota < (v & 15), xv, 0.0).astype(jnp.float32)

            def xzstep(i, carry, ob=ob):
                xyb[pl.ds(ob + i * L, L)] = jnp.zeros((L,), jnp.float32)
                return carry
            lax.fori_loop((v + 15) >> 4, HALF // L, xzstep, 0)
            for cg in range(8):
                poff = pl.multiple_of(
                    comp * (B * N) + (b >> 3) * (16 * 1024)
                    + (h * 8 + cg) * 1024 + (b & 7) * 128, 8)
                pltpu.async_copy(xyb.at[pl.ds(ob + cg * 128, 128)],
                                 pout.at[pl.ds(poff, 128)], sem_misc)
                deferred.append(
                    lambda ob=ob, cg=cg, poff=poff, pout=pout:
                    pltpu.make_async_copy(
                        xyb.at[pl.ds(ob + cg * 128, 128)],
                        pout.at[pl.ds(poff, 128)], sem_misc).wait())

        mb = si * HALF
        fill_mask(v, mb)
        for cg in range(8):
            moff = pl.multiple_of(
                (b >> 3) * (16 * 1024) + (h * 8 + cg) * 1024 + (b & 7) * 128, 8)
            pltpu.async_copy(mbuf.at[pl.ds(mb + cg * 128, 128)],
                             mout.at[pl.ds(moff, 128)], sem_misc)
            deferred.append(
                lambda mb=mb, cg=cg, moff=moff, mout=mout:
                pltpu.make_async_copy(
                    mbuf.at[pl.ds(mb + cg * 128, 128)],
                    mout.at[pl.ds(moff, 128)], sem_misc).wait())

    for wait_fn in deferred:
        wait_fn()


def _build(interpret=False):
    mesh = plsc.VectorSubcoreMesh(core_axis_name="c", subcore_axis_name="s",
                                  num_cores=2, num_subcores=16)
    out_type = (
        jax.ShapeDtypeStruct((B * N * 4, 128), jnp.float32),
        jax.ShapeDtypeStruct((3 * B * N,), jnp.float32),
        jax.ShapeDtypeStruct((B * N,), jnp.float32),
        jax.ShapeDtypeStruct((B * N * 4, 128), jnp.float32),
        jax.ShapeDtypeStruct((3 * B * N,), jnp.float32),
        jax.ShapeDtypeStruct((B * N,), jnp.float32),
    )
    scratch = [
        pltpu.VMEM((M,), jnp.int32),
        pltpu.VMEM((2 * TR, 128), jnp.float32),
        pltpu.VMEM((2 * TR,), jnp.int32),
        pltpu.VMEM((6 * XFETCH,), jnp.float32),
        pltpu.VMEM((3 * XSTRIDE,), jnp.float32),
        pltpu.VMEM((2 * HALF,), jnp.float32),
        pltpu.VMEM((128, 128), jnp.float32),
        pltpu.SemaphoreType.DMA,
        pltpu.SemaphoreType.DMA,
        pltpu.SemaphoreType.DMA,
        pltpu.SemaphoreType.DMA,
        pltpu.SemaphoreType.DMA,
    ]
    return pl.kernel(_sc_body, out_type=out_type, mesh=mesh,
                     scratch_types=scratch, interpret=interpret,
                     compiler_params=pltpu.CompilerParams(
                         use_tc_tiling_on_sc=False,
                         needs_layout_passes=False))


def kernel(bidx_src0, xyz_src0, feats_src0, bidx_src1, xyz_src1, feats_src1,
           batch_size, interpret=False):
    del batch_size
    zf = jnp.zeros((128, 128), jnp.float32)
    fn = _build(interpret)

    def ftab(feats):
        return feats.reshape(M // 8, 8, 4, 128).transpose(0, 2, 1, 3) \
                    .reshape(M // 8 * 32, 128)

    f0, p0, m0, f1, p1, m1 = fn(
        bidx_src0, ftab(feats_src0), xyz_src0.T.reshape(-1),
        bidx_src1, ftab(feats_src1), xyz_src1.T.reshape(-1), zf)

    def unmask(m):
        return m.reshape(2, 16, 8, 128).transpose(0, 2, 1, 3).reshape(B, N)

    def unpts(p):
        return p.reshape(3, 2, 16, 8, 128).transpose(1, 3, 2, 4, 0) \
                .reshape(B, N, 3)

    return (f0.reshape(B, N, C), unpts(p0), unmask(m0),
            f1.reshape(B, N, C), unpts(p1), unmask(m1))


# speedup vs baseline: 1.9525x; 1.9525x over previous
import jax
import jax.numpy as jnp
from jax import lax
from jax.experimental import pallas as pl
from jax.experimental.pallas import tpu as pltpu
from jax.experimental.pallas import tpu_sc as plsc

B = 16
M = 16384
C = 512
N = 2048
HALF = 1024
L = 16
CH = 64
NCH = HALF // CH
TR = 4 * CH
XFETCH = 1040
XSTRIDE = 1056

_SIZES_REM = [(1 << k) for k in range(5, -1, -1)]


def _sc_body(bidx0, ftab0, xyzt0, bidx1, ftab1, xyzt1, zf,
             f0, p0, m0, f1, p1, m1,
             bbuf, ring, ixbuf, xyb, xytmp, mbuf, zbuf,
             sem_in, semo0, semo1, sem_x, sem_misc):
    cax = lax.axis_index("c")
    sax = lax.axis_index("s")
    wid = cax * 16 + sax
    b = wid // 2
    h = wid % 2
    d0 = wid * HALF
    semos = (semo0, semo1)

    cz = pltpu.async_copy(zf, zbuf, sem_misc)
    cb = pltpu.async_copy(bidx0, bbuf, sem_misc)
    cz.wait()
    cb.wait()

    iota = lax.broadcasted_iota(jnp.int32, (L,), 0)

    def counts(bv):
        def step(i, carry):
            lo, hi = carry
            for u in range(4):
                x = bv[pl.ds((i * 4 + u) * L, L)]
                lo = lo + jnp.where(x < b, 1, 0).astype(jnp.int32)
                hi = hi + jnp.where(x <= b, 1, 0).astype(jnp.int32)
            return lo, hi
        z = jnp.zeros((L,), jnp.int32)
        lo, hi = lax.fori_loop(0, M // L // 4, step, (z, z))
        return jnp.sum(lo), jnp.sum(hi)

    seg0 = counts(bbuf)
    pltpu.sync_copy(bidx1, bbuf)
    seg1 = counts(bbuf)

    def fill_mask(v, mb):
        def step(i, carry):
            for u in range(4):
                j = i * 4 + u
                idx = j * L + iota
                mbuf[pl.ds(mb + j * L, L)] = jnp.where(
                    idx < v, 0.0, 1.0).astype(jnp.float32)
            return carry
        lax.fori_loop(0, HALF // L // 4, step, 0)

    def tiled_idx(i, g16):
        return ((i >> 3) << 5) + (g16 << 3) + (i & 7)

    deferred = []

    for si, (seg, ftab, xyzt, fout, pout, mout) in enumerate((
            (seg0, ftab0, xyzt0, f0, p0, m0),
            (seg1, ftab1, xyzt1, f1, p1, m1))):
        s0, e0 = seg
        v = jnp.clip(e0 - s0 - h * HALF, 0, HALF)
        srow = s0 + h * HALF
        rem = v & (CH - 1)
        nv = HALF - v

        sal = jnp.minimum((srow >> 3) << 3, M - XFETCH)
        sh = srow - sal
        xyz_ins = []
        for comp in range(3):
            a0 = pl.multiple_of(comp * M + sal, 8)
            d = pltpu.make_async_copy(
                xyzt.at[pl.ds(a0, XFETCH)],
                xytmp.at[pl.ds(comp * XSTRIDE, XFETCH)], sem_x)
            d.start()
            xyz_ins.append(d)

        seeds = []
        for t in range(16):
            j = t * L + iota
            seeds.append(tiled_idx(srow + ((j >> 5) << 3) + (j & 7),
                                   (j >> 3) & 3))

        def fire_chunk(c, slot):
            for t in range(16):
                ixbuf[pl.ds(slot * TR + t * L, L)] = seeds[t] + c * TR
            gs = []
            for hc in range(2):
                gs.append(pltpu.async_copy(
                    ftab.at[ixbuf.at[pl.ds(slot * TR + hc * 128, 128)]],
                    ring.at[pl.ds(slot * TR + hc * 128, 128)], sem_in))
            return gs

        for c in range(NCH):
            @pl.when(v >= (c + 1) * CH)
            def _(c=c):
                slot = c & 1
                if c >= 2:
                    pltpu.make_async_copy(
                        ring.at[pl.ds(slot * TR, TR)],
                        fout.at[pl.ds((d0 + (c - 2) * CH) * 4, TR)],
                        semos[slot]).wait()
                for g in fire_chunk(c, slot):
                    g.wait()
                pltpu.async_copy(
                    ring.at[pl.ds(slot * TR, TR)],
                    fout.at[pl.ds((d0 + c * CH) * 4, TR)], semos[slot])
        for slot in range(2):
            @pl.when(v >= (slot + 1) * CH)
            def _(slot=slot):
                pltpu.make_async_copy(
                    ring.at[pl.ds(slot * TR, TR)],
                    fout.at[pl.ds(d0 * 4, TR)], semos[slot]).wait()

        @pl.when(rem != 0)
        def _(v=v, rem=rem, srow=srow, fout=fout, ftab=ftab):
            cb0 = v - rem
            imax = srow + v - 1
            for t in range(16):
                j = t * L + iota
                i = jnp.minimum(srow + cb0 + ((j >> 5) << 3) + (j & 7), imax)
                ixbuf[pl.ds(t * L, L)] = tiled_idx(i, (j >> 3) & 3)
            for hc in range(2):
                pltpu.async_copy(
                    ftab.at[ixbuf.at[pl.ds(hc * 128, 128)]],
                    ring.at[pl.ds(hc * 128, 128)], sem_in).wait()
            rem7 = rem & 7
            nb = rem >> 3

            @pl.when(rem7 != 0)
            def _():
                zv = jnp.zeros((L,), jnp.float32)
                for g in range(4):
                    def zrow(r, carry, g=g):
                        p = nb * 32 + g * 8 + r
                        rows = jnp.broadcast_to(p, (L,))
                        for kk in range(8):
                            plsc.store_scatter(ring, [rows, kk * L + iota], zv)
                        return carry
                    lax.fori_loop(rem7, 8, zrow, 0)
            nbw = (rem + 7) >> 3
            for action in ("s", "w"):
                for sb in (8, 4, 2, 1):
                    k = sb.bit_length()
                    boff = (nbw >> k) << k

                    @pl.when((nbw & sb) != 0)
                    def _(boff=boff, sb=sb):
                        d = pltpu.make_async_copy(
                            ring.at[pl.ds(boff * 32, sb * 32)],
                            fout.at[pl.ds((d0 + cb0 + boff * 8) * 4, sb * 32)],
                            semo0)
                        d.start() if action == "s" else d.wait()

        def zfill(action, v=v, fout=fout):
            zs8 = ((v + 7) >> 3) << 3
            pad = (CH - (zs8 & (CH - 1))) & (CH - 1)
            for sz in (32, 16, 8):
                k = sz.bit_length()
                off = (pad >> k) << k

                @pl.when((pad & sz) != 0)
                def _(sz=sz, off=off):
                    d = pltpu.make_async_copy(
                        zbuf.at[pl.ds(0, sz * 4)],
                        fout.at[pl.ds((d0 + zs8 + off) * 4, sz * 4)],
                        sem_misc)
                    d.start() if action == "s" else d.wait()
            zs64 = zs8 + pad
            q = (HALF - zs64) >> 5

            def zstep(j, carry):
                d = pltpu.make_async_copy(
                    zbuf, fout.at[pl.ds((d0 + zs64) * 4 + j * 128, 128)],
                    sem_misc)
                d.start() if action == "s" else d.wait()
                return carry
            lax.fori_loop(0, q, zstep, 0)
        zfill("s")
        deferred.append(lambda zfill=zfill: zfill("w"))

        for d in xyz_ins:
            d.wait()
        for comp in range(3):
            tb = comp * XSTRIDE
            ob = (si * 3 + comp) * XFETCH

            def xstep(i, carry, tb=tb, ob=ob):
                xyb[pl.ds(ob + i * L, L)] = xytmp[pl.ds(tb + sh + i * L, L)]
                return carry
            lax.fori_loop(0, v >> 4, xstep, 0)
            fl = (v >> 4) << 4
            xv = xytmp[pl.ds(tb + sh + fl, L)]
            xyb[pl.ds(ob + fl, L)] = jnp.where(
                iota < (v & 15), xv, 0.0).astype(jnp.float32)

            def xzstep(i, carry, ob=ob):
                xyb[pl.ds(ob + i * L, L)] = jnp.zeros((L,), jnp.float32)
                return carry
            lax.fori_loop((v + 15) >> 4, HALF // L, xzstep, 0)
            for cg in range(8):
                poff = pl.multiple_of(
                    comp * (B * N) + (b >> 3) * (16 * 1024)
                    + (h * 8 + cg) * 1024 + (b & 7) * 128, 8)
                pltpu.async_copy(xyb.at[pl.ds(ob + cg * 128, 128)],
                                 pout.at[pl.ds(poff, 128)], sem_misc)
                deferred.append(
                    lambda ob=ob, cg=cg, poff=poff, pout=pout:
                    pltpu.make_async_copy(
                        xyb.at[pl.ds(ob + cg * 128, 128)],
                        pout.at[pl.ds(poff, 128)], sem_misc).wait())

        mb = si * HALF
        fill_mask(v, mb)
        for cg in range(8):
            moff = pl.multiple_of(
                (b >> 3) * (16 * 1024) + (h * 8 + cg) * 1024 + (b & 7) * 128, 8)
            pltpu.async_copy(mbuf.at[pl.ds(mb + cg * 128, 128)],
                             mout.at[pl.ds(moff, 128)], sem_misc)
            deferred.append(
                lambda mb=mb, cg=cg, moff=moff, mout=mout:
                pltpu.make_async_copy(
                    mbuf.at[pl.ds(mb + cg * 128, 128)],
                    mout.at[pl.ds(moff, 128)], sem_misc).wait())

    for wait_fn in deferred:
        wait_fn()


def _build(interpret=False):
    mesh = plsc.VectorSubcoreMesh(core_axis_name="c", subcore_axis_name="s",
                                  num_cores=2, num_subcores=16)
    out_type = (
        jax.ShapeDtypeStruct((B * N * 4, 128), jnp.float32),
        jax.ShapeDtypeStruct((3 * B * N,), jnp.float32),
        jax.ShapeDtypeStruct((B * N,), jnp.float32),
        jax.ShapeDtypeStruct((B * N * 4, 128), jnp.float32),
        jax.ShapeDtypeStruct((3 * B * N,), jnp.float32),
        jax.ShapeDtypeStruct((B * N,), jnp.float32),
    )
    scratch = [
        pltpu.VMEM((M,), jnp.int32),
        pltpu.VMEM((2 * TR, 128), jnp.float32),
        pltpu.VMEM((2 * TR,), jnp.int32),
        pltpu.VMEM((6 * XFETCH,), jnp.float32),
        pltpu.VMEM((3 * XSTRIDE,), jnp.float32),
        pltpu.VMEM((2 * HALF,), jnp.float32),
        pltpu.VMEM((128, 128), jnp.float32),
        pltpu.SemaphoreType.DMA,
        pltpu.SemaphoreType.DMA,
        pltpu.SemaphoreType.DMA,
        pltpu.SemaphoreType.DMA,
        pltpu.SemaphoreType.DMA,
    ]
    return pl.kernel(_sc_body, out_type=out_type, mesh=mesh,
                     scratch_types=scratch, interpret=interpret,
                     compiler_params=pltpu.CompilerParams(
                         use_tc_tiling_on_sc=False,
                         needs_layout_passes=False))


def kernel(bidx_src0, xyz_src0, feats_src0, bidx_src1, xyz_src1, feats_src1,
           batch_size, interpret=False):
    del batch_size
    zf = jnp.zeros((128, 128), jnp.float32)
    fn = _build(interpret)

    def ftab(feats):
        return feats.reshape(M // 8, 8, 4, 128).transpose(0, 2, 1, 3) \
                    .reshape(M // 8 * 32, 128)

    f0, p0, m0, f1, p1, m1 = fn(
        bidx_src0, ftab(feats_src0), xyz_src0.T.reshape(-1),
        bidx_src1, ftab(feats_src1), xyz_src1.T.reshape(-1), zf)

    def unmask(m):
        return m.reshape(2, 16, 8, 128).transpose(0, 2, 1, 3).reshape(B, N)

    def unpts(p):
        return p.reshape(3, 2, 16, 8, 128).transpose(1, 3, 2, 4, 0) \
                .reshape(B, N, 3)

    def unfeats(f):
        return f.reshape(B, N // 8, 4, 8, 128).transpose(0, 1, 3, 2, 4) \
                .reshape(B, N, C)

    return (unfeats(f0), unpts(p0), unmask(m0),
            unfeats(f1), unpts(p1), unmask(m1))


# speedup vs baseline: 2.1340x; 1.0930x over previous
import jax
import jax.numpy as jnp
from jax import lax
from jax.experimental import pallas as pl
from jax.experimental.pallas import tpu as pltpu
from jax.experimental.pallas import tpu_sc as plsc

B = 16
M = 16384
C = 512
N = 2048
HALF = 1024
L = 16
CH = 64
NCH = HALF // CH
TR = 4 * CH
XFETCH = 1040
XSTRIDE = 1056

_SIZES_REM = [(1 << k) for k in range(5, -1, -1)]


def _sc_body(bidx0, ftab0, xyzt0, bidx1, ftab1, xyzt1, zf,
             f0, p0, m0, f1, p1, m1,
             bbuf, bbuf2, ring, ixbuf, xyb, xytmp, mbuf, zbuf,
             sem_in, semo0, semo1, semg0, semg1, sem_x, sem_misc):
    cax = lax.axis_index("c")
    sax = lax.axis_index("s")
    wid = cax * 16 + sax
    b = wid // 2
    h = wid % 2
    d0 = wid * HALF
    semos = (semo0, semo1)
    semgs = (semg0, semg1)

    starts = [pltpu.async_copy(zf, zbuf, sem_misc),
              pltpu.async_copy(bidx0, bbuf, sem_misc),
              pltpu.async_copy(bidx1, bbuf2, sem_misc)]
    for d in starts:
        d.wait()

    iota = lax.broadcasted_iota(jnp.int32, (L,), 0)

    def counts(bv):
        def step(i, carry):
            lo, hi = carry
            for u in range(4):
                x = bv[pl.ds((i * 4 + u) * L, L)]
                lo = lo + jnp.where(x < b, 1, 0).astype(jnp.int32)
                hi = hi + jnp.where(x <= b, 1, 0).astype(jnp.int32)
            return lo, hi
        z = jnp.zeros((L,), jnp.int32)
        lo, hi = lax.fori_loop(0, M // L // 4, step, (z, z))
        return jnp.sum(lo), jnp.sum(hi)

    seg0 = counts(bbuf)
    seg1 = counts(bbuf2)

    def fill_mask(v, mb):
        def step(i, carry):
            for u in range(4):
                j = i * 4 + u
                idx = j * L + iota
                mbuf[pl.ds(mb + j * L, L)] = jnp.where(
                    idx < v, 0.0, 1.0).astype(jnp.float32)
            return carry
        lax.fori_loop(0, HALF // L // 4, step, 0)

    def tiled_idx(i, g16):
        return ((i >> 3) << 5) + (g16 << 3) + (i & 7)

    deferred = []

    for si, (seg, ftab, xyzt, fout, pout, mout) in enumerate((
            (seg0, ftab0, xyzt0, f0, p0, m0),
            (seg1, ftab1, xyzt1, f1, p1, m1))):
        s0, e0 = seg
        v = jnp.clip(e0 - s0 - h * HALF, 0, HALF)
        srow = s0 + h * HALF
        rem = v & (CH - 1)
        nv = HALF - v

        sal = jnp.minimum((srow >> 3) << 3, M - XFETCH)
        sh = srow - sal
        xyz_ins = []
        for comp in range(3):
            a0 = pl.multiple_of(comp * M + sal, 8)
            d = pltpu.make_async_copy(
                xyzt.at[pl.ds(a0, XFETCH)],
                xytmp.at[pl.ds(comp * XSTRIDE, XFETCH)], sem_x)
            d.start()
            xyz_ins.append(d)

        seeds = []
        for t in range(16):
            j = t * L + iota
            seeds.append(tiled_idx(srow + ((j >> 5) << 3) + (j & 7),
                                   (j >> 3) & 3))

        def fire_chunk(c, slot):
            for t in range(16):
                ixbuf[pl.ds(slot * TR + t * L, L)] = seeds[t] + c * TR
            for hc in range(2):
                pltpu.async_copy(
                    ftab.at[ixbuf.at[pl.ds(slot * TR + hc * 128, 128)]],
                    ring.at[pl.ds(slot * TR + hc * 128, 128)], semgs[slot])

        def wait_chunk(slot):
            for hc in range(2):
                pltpu.make_async_copy(
                    ftab.at[ixbuf.at[pl.ds(slot * TR + hc * 128, 128)]],
                    ring.at[pl.ds(slot * TR + hc * 128, 128)],
                    semgs[slot]).wait()

        @pl.when(v >= CH)
        def _():
            fire_chunk(0, 0)
        for c in range(NCH):
            @pl.when(v >= (c + 1) * CH)
            def _(c=c):
                slot = c & 1
                if c + 1 < NCH:
                    @pl.when(v >= (c + 2) * CH)
                    def _():
                        if c >= 1:
                            pltpu.make_async_copy(
                                ring.at[pl.ds((1 - slot) * TR, TR)],
                                fout.at[pl.ds((d0 + (c - 1) * CH) * 4, TR)],
                                semos[1 - slot]).wait()
                        fire_chunk(c + 1, 1 - slot)
                wait_chunk(slot)
                pltpu.async_copy(
                    ring.at[pl.ds(slot * TR, TR)],
                    fout.at[pl.ds((d0 + c * CH) * 4, TR)], semos[slot])
        for slot in range(2):
            @pl.when(v >= (slot + 1) * CH)
            def _(slot=slot):
                pltpu.make_async_copy(
                    ring.at[pl.ds(slot * TR, TR)],
                    fout.at[pl.ds(d0 * 4, TR)], semos[slot]).wait()

        @pl.when(rem != 0)
        def _(v=v, rem=rem, srow=srow, fout=fout, ftab=ftab):
            cb0 = v - rem
            imax = srow + v - 1
            for t in range(16):
                j = t * L + iota
                i = jnp.minimum(srow + cb0 + ((j >> 5) << 3) + (j & 7), imax)
                ixbuf[pl.ds(t * L, L)] = tiled_idx(i, (j >> 3) & 3)
            for hc in range(2):
                pltpu.async_copy(
                    ftab.at[ixbuf.at[pl.ds(hc * 128, 128)]],
                    ring.at[pl.ds(hc * 128, 128)], sem_in).wait()
            rem7 = rem & 7
            nb = rem >> 3

            @pl.when(rem7 != 0)
            def _():
                zv = jnp.zeros((L,), jnp.float32)
                for g in range(4):
                    def zrow(r, carry, g=g):
                        p = nb * 32 + g * 8 + r
                        rows = jnp.broadcast_to(p, (L,))
                        for kk in range(8):
                            plsc.store_scatter(ring, [rows, kk * L + iota], zv)
                        return carry
                    lax.fori_loop(rem7, 8, zrow, 0)
            nbw = (rem + 7) >> 3
            for action in ("s", "w"):
                for sb in (8, 4, 2, 1):
                    k = sb.bit_length()
                    boff = (nbw >> k) << k

                    @pl.when((nbw & sb) != 0)
                    def _(boff=boff, sb=sb):
                        d = pltpu.make_async_copy(
                            ring.at[pl.ds(boff * 32, sb * 32)],
                            fout.at[pl.ds((d0 + cb0 + boff * 8) * 4, sb * 32)],
                            semo0)
                        d.start() if action == "s" else d.wait()

        def zfill(action, v=v, fout=fout):
            zs8 = ((v + 7) >> 3) << 3
            pad = (CH - (zs8 & (CH - 1))) & (CH - 1)
            for sz in (32, 16, 8):
                k = sz.bit_length()
                off = (pad >> k) << k

                @pl.when((pad & sz) != 0)
                def _(sz=sz, off=off):
                    d = pltpu.make_async_copy(
                        zbuf.at[pl.ds(0, sz * 4)],
                        fout.at[pl.ds((d0 + zs8 + off) * 4, sz * 4)],
                        sem_misc)
                    d.start() if action == "s" else d.wait()
            zs64 = zs8 + pad
            q = (HALF - zs64) >> 5

            def zstep(j, carry):
                d = pltpu.make_async_copy(
                    zbuf, fout.at[pl.ds((d0 + zs64) * 4 + j * 128, 128)],
                    sem_misc)
                d.start() if action == "s" else d.wait()
                return carry
            lax.fori_loop(0, q, zstep, 0)
        zfill("s")
        deferred.append(lambda zfill=zfill: zfill("w"))

        for d in xyz_ins:
            d.wait()
        for comp in range(3):
            tb = comp * XSTRIDE
            ob = (si * 3 + comp) * XFETCH

            def xstep(i, carry, tb=tb, ob=ob):
                xyb[pl.ds(ob + i * L, L)] = xytmp[pl.ds(tb + sh + i * L, L)]
                return carry
            lax.fori_loop(0, v >> 4, xstep, 0)
            fl = (v >> 4) << 4
            xv = xytmp[pl.ds(tb + sh + fl, L)]
            xyb[pl.ds(ob + fl, L)] = jnp.where(
                iota < (v & 15), xv, 0.0).astype(jnp.float32)

            def xzstep(i, carry, ob=ob):
                xyb[pl.ds(ob + i * L, L)] = jnp.zeros((L,), jnp.float32)
                return carry
            lax.fori_loop((v + 15) >> 4, HALF // L, xzstep, 0)
            for cg in range(8):
                poff = pl.multiple_of(
                    comp * (B * N) + (b >> 3) * (16 * 1024)
                    + (h * 8 + cg) * 1024 + (b & 7) * 128, 8)
                pltpu.async_copy(xyb.at[pl.ds(ob + cg * 128, 128)],
                                 pout.at[pl.ds(poff, 128)], sem_misc)
                deferred.append(
                    lambda ob=ob, cg=cg, poff=poff, pout=pout:
                    pltpu.make_async_copy(
                        xyb.at[pl.ds(ob + cg * 128, 128)],
                        pout.at[pl.ds(poff, 128)], sem_misc).wait())

        mb = si * HALF
        fill_mask(v, mb)
        for cg in range(8):
            moff = pl.multiple_of(
                (b >> 3) * (16 * 1024) + (h * 8 + cg) * 1024 + (b & 7) * 128, 8)
            pltpu.async_copy(mbuf.at[pl.ds(mb + cg * 128, 128)],
                             mout.at[pl.ds(moff, 128)], sem_misc)
            deferred.append(
                lambda mb=mb, cg=cg, moff=moff, mout=mout:
                pltpu.make_async_copy(
                    mbuf.at[pl.ds(mb + cg * 128, 128)],
                    mout.at[pl.ds(moff, 128)], sem_misc).wait())

    for wait_fn in deferred:
        wait_fn()


def _build(interpret=False):
    mesh = plsc.VectorSubcoreMesh(core_axis_name="c", subcore_axis_name="s",
                                  num_cores=2, num_subcores=16)
    out_type = (
        jax.ShapeDtypeStruct((B * N * 4, 128), jnp.float32),
        jax.ShapeDtypeStruct((3 * B * N,), jnp.float32),
        jax.ShapeDtypeStruct((B * N,), jnp.float32),
        jax.ShapeDtypeStruct((B * N * 4, 128), jnp.float32),
        jax.ShapeDtypeStruct((3 * B * N,), jnp.float32),
        jax.ShapeDtypeStruct((B * N,), jnp.float32),
    )
    scratch = [
        pltpu.VMEM((M,), jnp.int32),
        pltpu.VMEM((M,), jnp.int32),
        pltpu.VMEM((2 * TR, 128), jnp.float32),
        pltpu.VMEM((2 * TR,), jnp.int32),
        pltpu.VMEM((6 * XFETCH,), jnp.float32),
        pltpu.VMEM((3 * XSTRIDE,), jnp.float32),
        pltpu.VMEM((2 * HALF,), jnp.float32),
        pltpu.VMEM((128, 128), jnp.float32),
        pltpu.SemaphoreType.DMA,
        pltpu.SemaphoreType.DMA,
        pltpu.SemaphoreType.DMA,
        pltpu.SemaphoreType.DMA,
        pltpu.SemaphoreType.DMA,
        pltpu.SemaphoreType.DMA,
        pltpu.SemaphoreType.DMA,
    ]
    return pl.kernel(_sc_body, out_type=out_type, mesh=mesh,
                     scratch_types=scratch, interpret=interpret,
                     compiler_params=pltpu.CompilerParams(
                         use_tc_tiling_on_sc=False,
                         needs_layout_passes=False))


def kernel(bidx_src0, xyz_src0, feats_src0, bidx_src1, xyz_src1, feats_src1,
           batch_size, interpret=False):
    del batch_size
    zf = jnp.zeros((128, 128), jnp.float32)
    fn = _build(interpret)

    def ftab(feats):
        return feats.reshape(M // 8, 8, 4, 128).transpose(0, 2, 1, 3) \
                    .reshape(M // 8 * 32, 128)

    f0, p0, m0, f1, p1, m1 = fn(
        bidx_src0, ftab(feats_src0), xyz_src0.T.reshape(-1),
        bidx_src1, ftab(feats_src1), xyz_src1.T.reshape(-1), zf)

    def unmask(m):
        return m.reshape(2, 16, 8, 128).transpose(0, 2, 1, 3).reshape(B, N)

    def unpts(p):
        return p.reshape(3, 2, 16, 8, 128).transpose(1, 3, 2, 4, 0) \
                .reshape(B, N, 3)

    def unfeats(f):
        return f.reshape(B, N // 8, 4, 8, 128).transpose(0, 1, 3, 2, 4) \
                .reshape(B, N, C)

    return (unfeats(f0), unpts(p0), unmask(m0),
            unfeats(f1), unpts(p1), unmask(m1))
